# baseline (speedup 1.0000x reference)
import functools

import jax
import jax.numpy as jnp
from jax import lax
from jax.experimental import pallas as pl
from jax.experimental.pallas import tpu as pltpu
from jax.experimental.pallas import tpu_sc as plsc

B, LT, LI = 64, 512, 100
VOCAB, MAXPOS, NTYPES, HIDDEN, IMG_DIM = 30522, 512, 2, 768, 2048
EPS = 1e-12

_NC, _NS = 2, 16
_NW = _NC * _NS
_HTOK = B * LT // 2
_PER_W = _HTOK // _NW
_CH = 64
_NCH = _PER_W // _CH


def _sc_word_gather(table, idx):
    mesh = plsc.VectorSubcoreMesh(core_axis_name="c", subcore_axis_name="s")

    @functools.partial(
        pl.kernel, mesh=mesh,
        out_type=jax.ShapeDtypeStruct((_HTOK, HIDDEN), jnp.float32),
        scratch_types=[
            pltpu.VMEM((_PER_W,), jnp.int32),
            pltpu.VMEM((2, _CH, HIDDEN), jnp.float32),
            pltpu.SemaphoreType.DMA((2,)),
            pltpu.SemaphoreType.DMA((2,)),
        ],
    )
    def k(table_hbm, idx_hbm, out_hbm, idx_v, rows_v, gsem, wsem):
        wid = lax.axis_index("s") * _NC + lax.axis_index("c")
        base = wid * _PER_W
        pltpu.sync_copy(idx_hbm.at[pl.ds(base, _PER_W)], idx_v)

        def g_args(c, b):
            return (table_hbm.at[idx_v.at[pl.ds(c * _CH, _CH)]],
                    rows_v.at[b], gsem.at[b])

        def w_args(c, b):
            return (rows_v.at[b], out_hbm.at[pl.ds(base + c * _CH, _CH)],
                    wsem.at[b])

        pltpu.async_copy(*g_args(0, 0))
        pltpu.async_copy(*g_args(1, 1))

        def body(j, carry):
            for b in range(2):
                c = 2 * j + b
                pltpu.make_async_copy(*g_args(c, b)).wait()
                pltpu.async_copy(*w_args(c, b))

            @pl.when(j < _NCH // 2 - 1)
            def _():
                for b in range(2):
                    c = 2 * j + b
                    pltpu.make_async_copy(*w_args(c, b)).wait()
                    pltpu.async_copy(*g_args(c + 2, b))

            return carry

        lax.fori_loop(0, _NCH // 2, body, 0)
        for b in range(2):
            pltpu.make_async_copy(*w_args(_NCH - 2 + b, b)).wait()

    return k(table, idx)


def _ln(x, g, b):
    mu = jnp.mean(x, axis=-1, keepdims=True)
    var = jnp.mean((x - mu) ** 2, axis=-1, keepdims=True)
    return (x - mu) * lax.rsqrt(var + EPS) * g + b


_TABN = 520


def _pos_type_lookup(ids_row, seg_row, tab):
    n = ids_row.shape[1]
    iota = lax.broadcasted_iota(jnp.int32, (_TABN, n), 0)
    ohc = ((iota == ids_row) | (iota == seg_row + MAXPOS)
           ).astype(jnp.bfloat16)
    return lax.dot_general(ohc, tab, (((0,), (0,)), ((), ())),
                           preferred_element_type=jnp.float32)


_SBLK = 4
_NTXT = LT // _SBLK
_NTXTH = _NTXT // 2
_NIMG = LI // _SBLK
_NOUT = _NTXT + _NIMG


def _tc_img_body(feat_ref, ipos_ref, iseg_ref, tab_ref, w_ref,
                 prm_ref, out_ref):
    img_b = prm_ref[2, :]
    imgln_g = prm_ref[3, :]
    imgln_b = prm_ref[4, :]
    imgln2_g = prm_ref[5, :]
    imgln2_b = prm_ref[6, :]

    i = pl.program_id(0)
    feat = feat_ref[...].reshape(_SBLK * B, IMG_DIM).astype(jnp.bfloat16)
    img = jnp.dot(feat, w_ref[...],
                  preferred_element_type=jnp.float32) + img_b
    img = _ln(img, imgln_g, imgln_b)
    pt = _pos_type_lookup(ipos_ref[pl.ds(i, 1), :], iseg_ref[pl.ds(i, 1), :],
                          tab_ref[...])
    out_ref[...] = _ln(img + pt, imgln2_g,
                       imgln2_b).reshape(_SBLK, B, HIDDEN)


def _make_txt_body(off):
    def _tc_txt_body(acc_ref, words_ref, tpos_ref, tseg_ref, tab_ref,
                     prm_ref, out_ref):
        del acc_ref
        j = pl.program_id(0)
        ln_g = prm_ref[0, :]
        ln_b = prm_ref[1, :]
        pt = _pos_type_lookup(tpos_ref[pl.ds(j + off, 1), :],
                              tseg_ref[pl.ds(j + off, 1), :], tab_ref[...])
        x = words_ref[...] + pt.reshape(_SBLK, B, HIDDEN)
        out_ref[...] = _ln(x, ln_g, ln_b)
    return _tc_txt_body


def _tc_txt_call(acc, words_h, tpos, tseg, tab, prm, off):
    return pl.pallas_call(
        _make_txt_body(off),
        grid=(_NTXTH,),
        in_specs=[
            pl.BlockSpec(memory_space=pltpu.MemorySpace.HBM),
            pl.BlockSpec((_SBLK, B, HIDDEN), lambda j: (j, 0, 0)),
            pl.BlockSpec((_NTXT, _SBLK * B), lambda j: (0, 0)),
            pl.BlockSpec((_NTXT, _SBLK * B), lambda j: (0, 0)),
            pl.BlockSpec((_TABN, HIDDEN), lambda j: (0, 0)),
            pl.BlockSpec((8, HIDDEN), lambda j: (0, 0)),
        ],
        out_specs=pl.BlockSpec((_SBLK, B, HIDDEN),
                               lambda j, o=off: (j + o, 0, 0)),
        out_shape=jax.ShapeDtypeStruct((LT + LI, B, HIDDEN), jnp.float32),
        input_output_aliases={0: 0},
        compiler_params=pltpu.CompilerParams(
            dimension_semantics=("arbitrary",)),
    )(acc, words_h, tpos, tseg, tab, prm)


def kernel(text_input_ids, text_position_ids, text_segment_ids, image_feat,
           image_position_ids, image_segment_ids, word_emb, pos_emb, type_emb,
           ln_g, ln_b, img_W, img_b, imgln_g, imgln_b, imgln2_g, imgln2_b):
    wid_t = text_input_ids.astype(jnp.int32).T.reshape(-1)
    tpos = text_position_ids.astype(jnp.int32).T.reshape(_NTXT, _SBLK * B)
    tseg = text_segment_ids.astype(jnp.int32).T.reshape(_NTXT, _SBLK * B)
    ipos = image_position_ids.astype(jnp.int32).T.reshape(_NIMG, _SBLK * B)
    iseg = image_segment_ids.astype(jnp.int32).T.reshape(_NIMG, _SBLK * B)
    feat_t = jnp.transpose(image_feat, (1, 0, 2))

    words1 = _sc_word_gather(word_emb, wid_t[:_HTOK]).reshape(
        LT // 2, B, HIDDEN)
    words2 = _sc_word_gather(word_emb, wid_t[_HTOK:]).reshape(
        LT // 2, B, HIDDEN)

    tab = jnp.concatenate(
        [pos_emb, type_emb,
         jnp.zeros((_TABN - MAXPOS - NTYPES, HIDDEN), jnp.float32)],
        axis=0).astype(jnp.bfloat16)
    prm = jnp.stack(
        [ln_g, ln_b, img_b, imgln_g, imgln_b, imgln2_g, imgln2_b,
         jnp.zeros((HIDDEN,), jnp.float32)], axis=0)

    acc = pl.pallas_call(
        _tc_img_body,
        grid=(_NIMG,),
        in_specs=[
            pl.BlockSpec((_SBLK, B, IMG_DIM), lambda i: (i, 0, 0)),
            pl.BlockSpec((_NIMG, _SBLK * B), lambda i: (0, 0)),
            pl.BlockSpec((_NIMG, _SBLK * B), lambda i: (0, 0)),
            pl.BlockSpec((_TABN, HIDDEN), lambda i: (0, 0)),
            pl.BlockSpec((IMG_DIM, HIDDEN), lambda i: (0, 0)),
            pl.BlockSpec((8, HIDDEN), lambda i: (0, 0)),
        ],
        out_specs=pl.BlockSpec((_SBLK, B, HIDDEN),
                               lambda i: (i + _NTXT, 0, 0)),
        out_shape=jax.ShapeDtypeStruct((LT + LI, B, HIDDEN), jnp.float32),
        compiler_params=pltpu.CompilerParams(
            dimension_semantics=("arbitrary",)),
    )(feat_t, ipos, iseg, tab, img_W.astype(jnp.bfloat16), prm)

    acc = _tc_txt_call(acc, words1, tpos, tseg, tab, prm, 0)
    acc = _tc_txt_call(acc, words2, tpos, tseg, tab, prm, _NTXTH)

    return jnp.transpose(acc, (1, 0, 2))


# speedup vs baseline: 1.2007x; 1.2007x over previous
import functools

import jax
import jax.numpy as jnp
from jax import lax
from jax.experimental import pallas as pl
from jax.experimental.pallas import tpu as pltpu
from jax.experimental.pallas import tpu_sc as plsc

B, LT, LI = 64, 512, 100
VOCAB, MAXPOS, NTYPES, HIDDEN, IMG_DIM = 30522, 512, 2, 768, 2048
EPS = 1e-12

_NC, _NS = 2, 16
_NW = _NC * _NS
_HTOK = B * LT // 2
_PER_W = _HTOK // _NW
_CH = 64
_NCH = _PER_W // _CH


def _sc_word_gather(table, idx):
    mesh = plsc.VectorSubcoreMesh(core_axis_name="c", subcore_axis_name="s")

    @functools.partial(
        pl.kernel, mesh=mesh,
        out_type=jax.ShapeDtypeStruct((_HTOK, HIDDEN), jnp.float32),
        scratch_types=[
            pltpu.VMEM((_PER_W,), jnp.int32),
            pltpu.VMEM((2, _CH, HIDDEN), jnp.float32),
            pltpu.SemaphoreType.DMA((2,)),
            pltpu.SemaphoreType.DMA((2,)),
        ],
    )
    def k(table_hbm, idx_hbm, out_hbm, idx_v, rows_v, gsem, wsem):
        wid = lax.axis_index("s") * _NC + lax.axis_index("c")
        base = wid * _PER_W
        pltpu.sync_copy(idx_hbm.at[pl.ds(base, _PER_W)], idx_v)

        def g_args(c, b):
            return (table_hbm.at[idx_v.at[pl.ds(c * _CH, _CH)]],
                    rows_v.at[b], gsem.at[b])

        def w_args(c, b):
            return (rows_v.at[b], out_hbm.at[pl.ds(base + c * _CH, _CH)],
                    wsem.at[b])

        pltpu.async_copy(*g_args(0, 0))
        pltpu.async_copy(*g_args(1, 1))

        def body(j, carry):
            for b in range(2):
                c = 2 * j + b
                pltpu.make_async_copy(*g_args(c, b)).wait()
                pltpu.async_copy(*w_args(c, b))

            @pl.when(j < _NCH // 2 - 1)
            def _():
                for b in range(2):
                    c = 2 * j + b
                    pltpu.make_async_copy(*w_args(c, b)).wait()
                    pltpu.async_copy(*g_args(c + 2, b))

            return carry

        lax.fori_loop(0, _NCH // 2, body, 0)
        for b in range(2):
            pltpu.make_async_copy(*w_args(_NCH - 2 + b, b)).wait()

    return k(table, idx)


def _ln(x, g, b):
    mu = jnp.mean(x, axis=-1, keepdims=True)
    var = jnp.mean((x - mu) ** 2, axis=-1, keepdims=True)
    return (x - mu) * lax.rsqrt(var + EPS) * g + b


_TABN = 520


def _pos_type_lookup(ids_row, seg_row, tab):
    n = ids_row.shape[1]
    iota = lax.broadcasted_iota(jnp.int32, (_TABN, n), 0)
    ohc = ((iota == ids_row) | (iota == seg_row + MAXPOS)
           ).astype(jnp.bfloat16)
    return lax.dot_general(ohc, tab, (((0,), (0,)), ((), ())),
                           preferred_element_type=jnp.float32)


_SBLK = 4
_NIMG = LI // _SBLK
_TBLK = 8
_NTXT = LT // _TBLK
_NTXTH = _NTXT // 2


def _tc_img_body(feat_ref, ipos_ref, iseg_ref, tab_ref, w_ref,
                 prm_ref, out_ref):
    img_b = prm_ref[2, :]
    imgln_g = prm_ref[3, :]
    imgln_b = prm_ref[4, :]
    imgln2_g = prm_ref[5, :]
    imgln2_b = prm_ref[6, :]

    i = pl.program_id(0)
    feat = feat_ref[...].reshape(_SBLK * B, IMG_DIM)
    img = jnp.dot(feat, w_ref[...],
                  preferred_element_type=jnp.float32) + img_b
    img = _ln(img, imgln_g, imgln_b)
    pt = _pos_type_lookup(ipos_ref[pl.ds(i, 1), :], iseg_ref[pl.ds(i, 1), :],
                          tab_ref[...])
    out_ref[...] = _ln(img + pt, imgln2_g,
                       imgln2_b).reshape(_SBLK, B, HIDDEN)


def _make_txt_body(off):
    def _tc_txt_body(acc_ref, words_ref, tpos_ref, tseg_ref, tab_ref,
                     prm_ref, out_ref):
        del acc_ref
        j = pl.program_id(0)
        ln_g = prm_ref[0, :]
        ln_b = prm_ref[1, :]
        pt = _pos_type_lookup(tpos_ref[pl.ds(j + off, 1), :],
                              tseg_ref[pl.ds(j + off, 1), :], tab_ref[...])
        x = words_ref[...] + pt.reshape(_TBLK, B, HIDDEN)
        out_ref[...] = _ln(x, ln_g, ln_b)
    return _tc_txt_body


def _tc_txt_call(acc, words_h, tpos, tseg, tab, prm, off):
    return pl.pallas_call(
        _make_txt_body(off),
        grid=(_NTXTH,),
        in_specs=[
            pl.BlockSpec(memory_space=pltpu.MemorySpace.HBM),
            pl.BlockSpec((_TBLK, B, HIDDEN), lambda j: (j, 0, 0)),
            pl.BlockSpec((_NTXT, _TBLK * B), lambda j: (0, 0)),
            pl.BlockSpec((_NTXT, _TBLK * B), lambda j: (0, 0)),
            pl.BlockSpec((_TABN, HIDDEN), lambda j: (0, 0)),
            pl.BlockSpec((8, HIDDEN), lambda j: (0, 0)),
        ],
        out_specs=pl.BlockSpec((_TBLK, B, HIDDEN),
                               lambda j, o=off: (j + o, 0, 0)),
        out_shape=jax.ShapeDtypeStruct((LT + LI, B, HIDDEN), jnp.float32),
        input_output_aliases={0: 0},
        compiler_params=pltpu.CompilerParams(
            dimension_semantics=("arbitrary",)),
    )(acc, words_h, tpos, tseg, tab, prm)


def kernel(text_input_ids, text_position_ids, text_segment_ids, image_feat,
           image_position_ids, image_segment_ids, word_emb, pos_emb, type_emb,
           ln_g, ln_b, img_W, img_b, imgln_g, imgln_b, imgln2_g, imgln2_b):
    wid_t = text_input_ids.astype(jnp.int32).T.reshape(-1)
    tpos = text_position_ids.astype(jnp.int32).T.reshape(_NTXT, _TBLK * B)
    tseg = text_segment_ids.astype(jnp.int32).T.reshape(_NTXT, _TBLK * B)
    ipos = image_position_ids.astype(jnp.int32).T.reshape(_NIMG, _SBLK * B)
    iseg = image_segment_ids.astype(jnp.int32).T.reshape(_NIMG, _SBLK * B)
    feat_t = jnp.transpose(image_feat, (1, 0, 2))

    words1 = _sc_word_gather(word_emb, wid_t[:_HTOK]).reshape(
        LT // 2, B, HIDDEN)
    words2 = _sc_word_gather(word_emb, wid_t[_HTOK:]).reshape(
        LT // 2, B, HIDDEN)

    tab = jnp.concatenate(
        [pos_emb, type_emb,
         jnp.zeros((_TABN - MAXPOS - NTYPES, HIDDEN), jnp.float32)],
        axis=0).astype(jnp.bfloat16)
    prm = jnp.stack(
        [ln_g, ln_b, img_b, imgln_g, imgln_b, imgln2_g, imgln2_b,
         jnp.zeros((HIDDEN,), jnp.float32)], axis=0)

    acc = pl.pallas_call(
        _tc_img_body,
        grid=(_NIMG,),
        in_specs=[
            pl.BlockSpec((_SBLK, B, IMG_DIM), lambda i: (i, 0, 0)),
            pl.BlockSpec((_NIMG, _SBLK * B), lambda i: (0, 0)),
            pl.BlockSpec((_NIMG, _SBLK * B), lambda i: (0, 0)),
            pl.BlockSpec((_TABN, HIDDEN), lambda i: (0, 0)),
            pl.BlockSpec((IMG_DIM, HIDDEN), lambda i: (0, 0)),
            pl.BlockSpec((8, HIDDEN), lambda i: (0, 0)),
        ],
        out_specs=pl.BlockSpec((_SBLK, B, HIDDEN),
                               lambda i: (i + LT // _SBLK, 0, 0)),
        out_shape=jax.ShapeDtypeStruct((LT + LI, B, HIDDEN), jnp.float32),
        compiler_params=pltpu.CompilerParams(
            dimension_semantics=("arbitrary",)),
    )(feat_t, ipos, iseg, tab, img_W, prm)

    acc = _tc_txt_call(acc, words1, tpos, tseg, tab, prm, 0)
    acc = _tc_txt_call(acc, words2, tpos, tseg, tab, prm, _NTXTH)

    return jnp.transpose(acc, (1, 0, 2))


# speedup vs baseline: 1.3293x; 1.1072x over previous
import functools

import jax
import jax.numpy as jnp
from jax import lax
from jax.experimental import pallas as pl
from jax.experimental.pallas import tpu as pltpu
from jax.experimental.pallas import tpu_sc as plsc

B, LT, LI = 64, 512, 100
VOCAB, MAXPOS, NTYPES, HIDDEN, IMG_DIM = 30522, 512, 2, 768, 2048
EPS = 1e-12

_NC, _NS = 2, 16
_NW = _NC * _NS
_HTOK = B * LT // 2
_PER_W = _HTOK // _NW
_CH = 64
_NCH = _PER_W // _CH


def _sc_word_gather(table, idx):
    mesh = plsc.VectorSubcoreMesh(core_axis_name="c", subcore_axis_name="s")

    @functools.partial(
        pl.kernel, mesh=mesh,
        out_type=jax.ShapeDtypeStruct((_HTOK, HIDDEN), jnp.float32),
        scratch_types=[
            pltpu.VMEM((_PER_W,), jnp.int32),
            pltpu.VMEM((2, _CH, HIDDEN), jnp.float32),
            pltpu.SemaphoreType.DMA((2,)),
            pltpu.SemaphoreType.DMA((2,)),
        ],
    )
    def k(table_hbm, idx_hbm, out_hbm, idx_v, rows_v, gsem, wsem):
        wid = lax.axis_index("s") * _NC + lax.axis_index("c")
        base = wid * _PER_W
        pltpu.sync_copy(idx_hbm.at[pl.ds(base, _PER_W)], idx_v)

        def g_args(c, b):
            return (table_hbm.at[idx_v.at[pl.ds(c * _CH, _CH)]],
                    rows_v.at[b], gsem.at[b])

        def w_args(c, b):
            return (rows_v.at[b], out_hbm.at[pl.ds(base + c * _CH, _CH)],
                    wsem.at[b])

        pltpu.async_copy(*g_args(0, 0))
        pltpu.async_copy(*g_args(1, 1))

        def body(j, carry):
            for b in range(2):
                c = 2 * j + b
                pltpu.make_async_copy(*g_args(c, b)).wait()
                pltpu.async_copy(*w_args(c, b))

            @pl.when(j < _NCH // 2 - 1)
            def _():
                for b in range(2):
                    c = 2 * j + b
                    pltpu.make_async_copy(*w_args(c, b)).wait()
                    pltpu.async_copy(*g_args(c + 2, b))

            return carry

        lax.fori_loop(0, _NCH // 2, body, 0)
        for b in range(2):
            pltpu.make_async_copy(*w_args(_NCH - 2 + b, b)).wait()

    return k(table, idx)


def _ln(x, g, b):
    mu = jnp.mean(x, axis=-1, keepdims=True)
    var = jnp.mean((x - mu) ** 2, axis=-1, keepdims=True)
    return (x - mu) * lax.rsqrt(var + EPS) * g + b


_TABN = 520


def _pos_type_lookup(ids_row, seg_row, tab):
    n = ids_row.shape[1]
    iota = lax.broadcasted_iota(jnp.int32, (_TABN, n), 0)
    ohc = ((iota == ids_row) | (iota == seg_row + MAXPOS)
           ).astype(jnp.bfloat16)
    return lax.dot_general(ohc, tab, (((0,), (0,)), ((), ())),
                           preferred_element_type=jnp.float32)


_SBLK = 8
_NIMG = -(-LI // _SBLK)
_IPAD = _NIMG * _SBLK * B
_TBLK = 16
_NTXT = LT // _TBLK
_NTXTH = _NTXT // 2


def _tc_img_body(feat_ref, ipos_ref, iseg_ref, tab_ref, w_ref,
                 prm_ref, out_ref):
    img_b = prm_ref[2, :]
    imgln_g = prm_ref[3, :]
    imgln_b = prm_ref[4, :]
    imgln2_g = prm_ref[5, :]
    imgln2_b = prm_ref[6, :]

    i = pl.program_id(0)
    feat = feat_ref[...].reshape(_SBLK * B, IMG_DIM)
    img = jnp.dot(feat, w_ref[...],
                  preferred_element_type=jnp.float32) + img_b
    img = _ln(img, imgln_g, imgln_b)
    pt = _pos_type_lookup(ipos_ref[pl.ds(i, 1), :], iseg_ref[pl.ds(i, 1), :],
                          tab_ref[...])
    out_ref[...] = _ln(img + pt, imgln2_g,
                       imgln2_b).reshape(_SBLK, B, HIDDEN)


def _make_txt_body(off):
    def _tc_txt_body(acc_ref, words_ref, tpos_ref, tseg_ref, tab_ref,
                     prm_ref, out_ref):
        del acc_ref
        j = pl.program_id(0)
        ln_g = prm_ref[0, :]
        ln_b = prm_ref[1, :]
        pt = _pos_type_lookup(tpos_ref[pl.ds(j + off, 1), :],
                              tseg_ref[pl.ds(j + off, 1), :], tab_ref[...])
        x = words_ref[...] + pt.reshape(_TBLK, B, HIDDEN)
        out_ref[...] = _ln(x, ln_g, ln_b)
    return _tc_txt_body


def _tc_txt_call(acc, words_h, tpos, tseg, tab, prm, off):
    return pl.pallas_call(
        _make_txt_body(off),
        grid=(_NTXTH,),
        in_specs=[
            pl.BlockSpec(memory_space=pltpu.MemorySpace.HBM),
            pl.BlockSpec((_TBLK, B, HIDDEN), lambda j: (j, 0, 0)),
            pl.BlockSpec((_NTXT, _TBLK * B), lambda j: (0, 0)),
            pl.BlockSpec((_NTXT, _TBLK * B), lambda j: (0, 0)),
            pl.BlockSpec((_TABN, HIDDEN), lambda j: (0, 0)),
            pl.BlockSpec((8, HIDDEN), lambda j: (0, 0)),
        ],
        out_specs=pl.BlockSpec((_TBLK, B, HIDDEN),
                               lambda j, o=off: (j + o, 0, 0)),
        out_shape=jax.ShapeDtypeStruct((LT + LI, B, HIDDEN), jnp.float32),
        input_output_aliases={0: 0},
        compiler_params=pltpu.CompilerParams(
            dimension_semantics=("arbitrary",)),
    )(acc, words_h, tpos, tseg, tab, prm)


def kernel(text_input_ids, text_position_ids, text_segment_ids, image_feat,
           image_position_ids, image_segment_ids, word_emb, pos_emb, type_emb,
           ln_g, ln_b, img_W, img_b, imgln_g, imgln_b, imgln2_g, imgln2_b):
    wid_t = text_input_ids.astype(jnp.int32).T.reshape(-1)
    tpos = text_position_ids.astype(jnp.int32).T.reshape(_NTXT, _TBLK * B)
    tseg = text_segment_ids.astype(jnp.int32).T.reshape(_NTXT, _TBLK * B)
    ipos = jnp.pad(image_position_ids.astype(jnp.int32).T.reshape(-1),
                   (0, _IPAD - LI * B)).reshape(_NIMG, _SBLK * B)
    iseg = jnp.pad(image_segment_ids.astype(jnp.int32).T.reshape(-1),
                   (0, _IPAD - LI * B)).reshape(_NIMG, _SBLK * B)
    feat_t = jnp.transpose(image_feat, (1, 0, 2))

    words1 = _sc_word_gather(word_emb, wid_t[:_HTOK]).reshape(
        LT // 2, B, HIDDEN)
    words2 = _sc_word_gather(word_emb, wid_t[_HTOK:]).reshape(
        LT // 2, B, HIDDEN)

    tab = jnp.concatenate(
        [pos_emb, type_emb,
         jnp.zeros((_TABN - MAXPOS - NTYPES, HIDDEN), jnp.float32)],
        axis=0).astype(jnp.bfloat16)
    prm = jnp.stack(
        [ln_g, ln_b, img_b, imgln_g, imgln_b, imgln2_g, imgln2_b,
         jnp.zeros((HIDDEN,), jnp.float32)], axis=0)

    acc = pl.pallas_call(
        _tc_img_body,
        grid=(_NIMG,),
        in_specs=[
            pl.BlockSpec((_SBLK, B, IMG_DIM), lambda i: (i, 0, 0)),
            pl.BlockSpec((_NIMG, _SBLK * B), lambda i: (0, 0)),
            pl.BlockSpec((_NIMG, _SBLK * B), lambda i: (0, 0)),
            pl.BlockSpec((_TABN, HIDDEN), lambda i: (0, 0)),
            pl.BlockSpec((IMG_DIM, HIDDEN), lambda i: (0, 0)),
            pl.BlockSpec((8, HIDDEN), lambda i: (0, 0)),
        ],
        out_specs=pl.BlockSpec((_SBLK, B, HIDDEN),
                               lambda i: (i + LT // _SBLK, 0, 0)),
        out_shape=jax.ShapeDtypeStruct((LT + LI, B, HIDDEN), jnp.float32),
        compiler_params=pltpu.CompilerParams(
            dimension_semantics=("arbitrary",)),
    )(feat_t, ipos, iseg, tab, img_W, prm)

    acc = _tc_txt_call(acc, words1, tpos, tseg, tab, prm, 0)
    acc = _tc_txt_call(acc, words2, tpos, tseg, tab, prm, _NTXTH)

    return jnp.transpose(acc, (1, 0, 2))


# speedup vs baseline: 1.3622x; 1.0247x over previous
import functools

import jax
import jax.numpy as jnp
from jax import lax
from jax.experimental import pallas as pl
from jax.experimental.pallas import tpu as pltpu
from jax.experimental.pallas import tpu_sc as plsc

B, LT, LI = 64, 512, 100
VOCAB, MAXPOS, NTYPES, HIDDEN, IMG_DIM = 30522, 512, 2, 768, 2048
EPS = 1e-12

_NC, _NS = 2, 16
_NW = _NC * _NS
_HTOK = B * LT // 2
_PER_W = _HTOK // _NW
_CH = 64
_NCH = _PER_W // _CH


def _sc_word_gather(table, idx):
    mesh = plsc.VectorSubcoreMesh(core_axis_name="c", subcore_axis_name="s")

    @functools.partial(
        pl.kernel, mesh=mesh,
        out_type=jax.ShapeDtypeStruct((_HTOK, HIDDEN), jnp.float32),
        scratch_types=[
            pltpu.VMEM((_PER_W,), jnp.int32),
            pltpu.VMEM((2, _CH, HIDDEN), jnp.float32),
            pltpu.SemaphoreType.DMA((2,)),
            pltpu.SemaphoreType.DMA((2,)),
        ],
    )
    def k(table_hbm, idx_hbm, out_hbm, idx_v, rows_v, gsem, wsem):
        wid = lax.axis_index("s") * _NC + lax.axis_index("c")
        base = wid * _PER_W
        pltpu.sync_copy(idx_hbm.at[pl.ds(base, _PER_W)], idx_v)

        def g_args(c, b):
            return (table_hbm.at[idx_v.at[pl.ds(c * _CH, _CH)]],
                    rows_v.at[b], gsem.at[b])

        def w_args(c, b):
            return (rows_v.at[b], out_hbm.at[pl.ds(base + c * _CH, _CH)],
                    wsem.at[b])

        pltpu.async_copy(*g_args(0, 0))
        pltpu.async_copy(*g_args(1, 1))

        def body(j, carry):
            for b in range(2):
                c = 2 * j + b
                pltpu.make_async_copy(*g_args(c, b)).wait()
                pltpu.async_copy(*w_args(c, b))

            @pl.when(j < _NCH // 2 - 1)
            def _():
                for b in range(2):
                    c = 2 * j + b
                    pltpu.make_async_copy(*w_args(c, b)).wait()
                    pltpu.async_copy(*g_args(c + 2, b))

            return carry

        lax.fori_loop(0, _NCH // 2, body, 0)
        for b in range(2):
            pltpu.make_async_copy(*w_args(_NCH - 2 + b, b)).wait()

    return k(table, idx)


def _ln(x, g, b):
    mu = jnp.mean(x, axis=-1, keepdims=True)
    var = jnp.mean((x - mu) ** 2, axis=-1, keepdims=True)
    return (x - mu) * lax.rsqrt(var + EPS) * g + b


_TABN = 520


def _pos_type_lookup(ids_row, seg_row, tab):
    n = ids_row.shape[1]
    iota = lax.broadcasted_iota(jnp.int32, (_TABN, n), 0)
    ohc = ((iota == ids_row) | (iota == seg_row + MAXPOS)
           ).astype(jnp.bfloat16)
    return lax.dot_general(ohc, tab, (((0,), (0,)), ((), ())),
                           preferred_element_type=jnp.float32)


_SBLK = 16
_NIMG = -(-LI // _SBLK)
_IPAD = _NIMG * _SBLK * B
_TBLK = 32
_NTXT = LT // _TBLK
_NTXTH = _NTXT // 2


def _tc_img_body(feat_ref, ipos_ref, iseg_ref, tab_ref, w_ref,
                 prm_ref, out_ref):
    img_b = prm_ref[2, :]
    imgln_g = prm_ref[3, :]
    imgln_b = prm_ref[4, :]
    imgln2_g = prm_ref[5, :]
    imgln2_b = prm_ref[6, :]

    i = pl.program_id(0)
    feat = feat_ref[...].reshape(_SBLK * B, IMG_DIM)
    img = jnp.dot(feat, w_ref[...],
                  preferred_element_type=jnp.float32) + img_b
    img = _ln(img, imgln_g, imgln_b)
    pt = _pos_type_lookup(ipos_ref[pl.ds(i, 1), :], iseg_ref[pl.ds(i, 1), :],
                          tab_ref[...])
    out_ref[...] = _ln(img + pt, imgln2_g,
                       imgln2_b).reshape(_SBLK, B, HIDDEN)


def _make_txt_body(off):
    def _tc_txt_body(acc_ref, words_ref, tpos_ref, tseg_ref, tab_ref,
                     prm_ref, out_ref):
        del acc_ref
        j = pl.program_id(0)
        ln_g = prm_ref[0, :]
        ln_b = prm_ref[1, :]
        pt = _pos_type_lookup(tpos_ref[pl.ds(j + off, 1), :],
                              tseg_ref[pl.ds(j + off, 1), :], tab_ref[...])
        x = words_ref[...] + pt.reshape(_TBLK, B, HIDDEN)
        out_ref[...] = _ln(x, ln_g, ln_b)
    return _tc_txt_body


def _tc_txt_call(acc, words_h, tpos, tseg, tab, prm, off):
    return pl.pallas_call(
        _make_txt_body(off),
        grid=(_NTXTH,),
        in_specs=[
            pl.BlockSpec(memory_space=pltpu.MemorySpace.HBM),
            pl.BlockSpec((_TBLK, B, HIDDEN), lambda j: (j, 0, 0)),
            pl.BlockSpec((_NTXT, _TBLK * B), lambda j: (0, 0)),
            pl.BlockSpec((_NTXT, _TBLK * B), lambda j: (0, 0)),
            pl.BlockSpec((_TABN, HIDDEN), lambda j: (0, 0)),
            pl.BlockSpec((8, HIDDEN), lambda j: (0, 0)),
        ],
        out_specs=pl.BlockSpec((_TBLK, B, HIDDEN),
                               lambda j, o=off: (j + o, 0, 0)),
        out_shape=jax.ShapeDtypeStruct((LT + LI, B, HIDDEN), jnp.float32),
        input_output_aliases={0: 0},
        compiler_params=pltpu.CompilerParams(
            dimension_semantics=("arbitrary",)),
    )(acc, words_h, tpos, tseg, tab, prm)


def kernel(text_input_ids, text_position_ids, text_segment_ids, image_feat,
           image_position_ids, image_segment_ids, word_emb, pos_emb, type_emb,
           ln_g, ln_b, img_W, img_b, imgln_g, imgln_b, imgln2_g, imgln2_b):
    wid_t = text_input_ids.astype(jnp.int32).T.reshape(-1)
    tpos = text_position_ids.astype(jnp.int32).T.reshape(_NTXT, _TBLK * B)
    tseg = text_segment_ids.astype(jnp.int32).T.reshape(_NTXT, _TBLK * B)
    ipos = jnp.pad(image_position_ids.astype(jnp.int32).T.reshape(-1),
                   (0, _IPAD - LI * B)).reshape(_NIMG, _SBLK * B)
    iseg = jnp.pad(image_segment_ids.astype(jnp.int32).T.reshape(-1),
                   (0, _IPAD - LI * B)).reshape(_NIMG, _SBLK * B)
    feat_t = jnp.transpose(image_feat, (1, 0, 2))

    words1 = _sc_word_gather(word_emb, wid_t[:_HTOK]).reshape(
        LT // 2, B, HIDDEN)
    words2 = _sc_word_gather(word_emb, wid_t[_HTOK:]).reshape(
        LT // 2, B, HIDDEN)

    tab = jnp.concatenate(
        [pos_emb, type_emb,
         jnp.zeros((_TABN - MAXPOS - NTYPES, HIDDEN), jnp.float32)],
        axis=0).astype(jnp.bfloat16)
    prm = jnp.stack(
        [ln_g, ln_b, img_b, imgln_g, imgln_b, imgln2_g, imgln2_b,
         jnp.zeros((HIDDEN,), jnp.float32)], axis=0)

    acc = pl.pallas_call(
        _tc_img_body,
        grid=(_NIMG,),
        in_specs=[
            pl.BlockSpec((_SBLK, B, IMG_DIM), lambda i: (i, 0, 0)),
            pl.BlockSpec((_NIMG, _SBLK * B), lambda i: (0, 0)),
            pl.BlockSpec((_NIMG, _SBLK * B), lambda i: (0, 0)),
            pl.BlockSpec((_TABN, HIDDEN), lambda i: (0, 0)),
            pl.BlockSpec((IMG_DIM, HIDDEN), lambda i: (0, 0)),
            pl.BlockSpec((8, HIDDEN), lambda i: (0, 0)),
        ],
        out_specs=pl.BlockSpec((_SBLK, B, HIDDEN),
                               lambda i: (i + LT // _SBLK, 0, 0)),
        out_shape=jax.ShapeDtypeStruct((LT + LI, B, HIDDEN), jnp.float32),
        compiler_params=pltpu.CompilerParams(
            dimension_semantics=("arbitrary",)),
    )(feat_t, ipos, iseg, tab, img_W, prm)

    acc = _tc_txt_call(acc, words1, tpos, tseg, tab, prm, 0)
    acc = _tc_txt_call(acc, words2, tpos, tseg, tab, prm, _NTXTH)

    return jnp.transpose(acc, (1, 0, 2))
